# Initial kernel scaffold; baseline (speedup 1.0000x reference)
import jax
import jax.numpy as jnp
from jax.experimental import pallas as pl


def kernel(node_features, edge_index, edge_features, Wu, bu, a_w, We, be, Wm):
    raise NotImplementedError("write your pallas kernel here")


# speedup vs baseline: 2.6312x; 2.6312x over previous
import functools

import jax
import jax.numpy as jnp
from jax import lax
from jax.experimental import pallas as pl
from jax.experimental.pallas import tpu as pltpu
from jax.experimental.pallas import tpu_sc as plsc

N = 10000
E = 320000
D = 128
DE = 16
ALPHA = 0.3

NC = 2
NS = 16
L = 16
NW = NC * NS
EPW = E // NW
CHUNK = 128
NCHT = E // CHUNK
KMAX = -(-NCHT // NW)
NPAD = 10240
RPT = NPAD // NS

NODE_BLK = 1000
EDGE_BLK = 3200


def _mesh():
    return plsc.VectorSubcoreMesh(
        core_axis_name="c", subcore_axis_name="s", num_cores=NC, num_subcores=NS
    )


def _node_body(nf_ref, wu_ref, bu_ref, a1_ref, a2_ref, sab_ref):
    h = jnp.dot(nf_ref[...], wu_ref[...], preferred_element_type=jnp.float32)
    h = h + bu_ref[...]
    h = jnp.where(h > 0, h, ALPHA * h)
    sa = jnp.sum(h * a1_ref[...], axis=1)
    sb = jnp.sum(h * a2_ref[...], axis=1)
    sab_ref[...] = jnp.stack([sa, sb])[None]


def _node_stage(nf, Wu, bu2, a1, a2):
    return pl.pallas_call(
        _node_body,
        grid=(N // NODE_BLK,),
        in_specs=[
            pl.BlockSpec((NODE_BLK, D), lambda i: (i, 0)),
            pl.BlockSpec((D, D), lambda i: (0, 0)),
            pl.BlockSpec((1, D), lambda i: (0, 0)),
            pl.BlockSpec((1, D), lambda i: (0, 0)),
            pl.BlockSpec((1, D), lambda i: (0, 0)),
        ],
        out_specs=pl.BlockSpec((1, 2, NODE_BLK), lambda i: (i, 0, 0)),
        out_shape=jax.ShapeDtypeStruct((N // NODE_BLK, 2, NODE_BLK), jnp.float32),
    )(nf, Wu, bu2, a1, a2)


def _att_stage(sa, sb, i0, i1):
    @functools.partial(
        pl.kernel,
        out_type=jax.ShapeDtypeStruct((E,), jnp.float32),
        mesh=_mesh(),
        compiler_params=pltpu.CompilerParams(needs_layout_passes=False),
        scratch_types=[
            pltpu.VMEM((N,), jnp.float32),
            pltpu.VMEM((N,), jnp.float32),
            pltpu.VMEM((EPW,), jnp.int32),
            pltpu.VMEM((EPW,), jnp.int32),
            pltpu.VMEM((EPW,), jnp.float32),
        ],
    )
    def k(sa_hbm, sb_hbm, i0_hbm, i1_hbm, att_hbm, sa_v, sb_v, i0_v, i1_v, att_v):
        wid = lax.axis_index("s") * NC + lax.axis_index("c")
        base = wid * EPW
        pltpu.sync_copy(sa_hbm, sa_v)
        pltpu.sync_copy(sb_hbm, sb_v)
        pltpu.sync_copy(i0_hbm.at[pl.ds(base, EPW)], i0_v)
        pltpu.sync_copy(i1_hbm.at[pl.ds(base, EPW)], i1_v)

        def body(kk, carry):
            o = kk * L
            va = plsc.load_gather(sa_v, [i1_v[pl.ds(o, L)]])
            vb = plsc.load_gather(sb_v, [i0_v[pl.ds(o, L)]])
            att_v[pl.ds(o, L)] = va + vb
            return carry

        lax.fori_loop(0, EPW // L, body, 0)
        pltpu.sync_copy(att_v, att_hbm.at[pl.ds(base, EPW)])

    return k(sa, sb, i0, i1)


def _edge_body(att_ref, eft_ref, wet_ref, bet_ref, wmt_ref, qt_ref):
    ut = jnp.dot(wet_ref[...], eft_ref[...], preferred_element_type=jnp.float32)
    ut = ut + bet_ref[...]
    xt = ut * att_ref[...]
    xt = xt - jnp.max(xt, axis=0, keepdims=True)
    p = jnp.exp(xt)
    p = p / jnp.sum(p, axis=0, keepdims=True)
    yt = jnp.dot(wmt_ref[...], p, preferred_element_type=jnp.float32)
    yt = yt - jnp.max(yt, axis=0, keepdims=True)
    q = jnp.exp(yt)
    qt_ref[...] = q / jnp.sum(q, axis=0, keepdims=True)


def _edge_stage(att2d, eft, wet, bet, wmt):
    return pl.pallas_call(
        _edge_body,
        grid=(E // EDGE_BLK,),
        in_specs=[
            pl.BlockSpec((1, EDGE_BLK), lambda i: (0, i)),
            pl.BlockSpec((DE, EDGE_BLK), lambda i: (0, i)),
            pl.BlockSpec((D, DE), lambda i: (0, 0)),
            pl.BlockSpec((D, 1), lambda i: (0, 0)),
            pl.BlockSpec((D, D), lambda i: (0, 0)),
        ],
        out_specs=pl.BlockSpec((D, EDGE_BLK), lambda i: (0, i)),
        out_shape=jax.ShapeDtypeStruct((D, E), jnp.float32),
    )(att2d, eft, wet, bet, wmt)


def _scatter_stage(qt, nf, i0, i1, zc):
    @functools.partial(
        pl.kernel,
        out_type=jax.ShapeDtypeStruct((NC, N, D), jnp.float32),
        mesh=_mesh(),
        compiler_params=pltpu.CompilerParams(needs_layout_passes=False),
        scratch_types=[
            pltpu.VMEM((CHUNK,), jnp.int32),
            pltpu.VMEM((CHUNK,), jnp.int32),
            pltpu.VMEM((CHUNK, D), jnp.float32),
            pltpu.VMEM((D, CHUNK), jnp.float32),
            pltpu.VMEM_SHARED((NPAD, D), jnp.float32),
            pltpu.SemaphoreType.DMA,
        ],
    )
    def k(qt_hbm, nf_hbm, i0_hbm, i1_hbm, zc_hbm, out_hbm,
          i0_v, i1_v, nbr_v, qt_v, acc_sh, sem):
        c = lax.axis_index("c")
        s = lax.axis_index("s")
        wid = s * NC + c
        row0 = pl.multiple_of(s * RPT, RPT)
        pltpu.sync_copy(zc_hbm, acc_sh.at[pl.ds(row0, RPT)])
        plsc.subcore_barrier()

        iot = lax.iota(jnp.int32, L)

        def chunk_body(g, carry):
            ch = g * NW + wid

            @pl.when(ch < NCHT)
            def _():
                base = pl.multiple_of(ch * CHUNK, CHUNK)
                pltpu.sync_copy(i0_hbm.at[pl.ds(base, CHUNK)], i0_v)
                pltpu.sync_copy(i1_hbm.at[pl.ds(base, CHUNK)], i1_v)
                pltpu.async_copy(nf_hbm.at[i0_v], nbr_v, sem).wait()
                pltpu.sync_copy(qt_hbm.at[:, pl.ds(base, CHUNK)], qt_v)

                def edge_body(e, carry2):
                    col = jnp.zeros((L,), jnp.int32) + e
                    for j in range(D // L):
                        qv = plsc.load_gather(qt_v, [iot + j * L, col])
                        nv = nbr_v[e, pl.ds(j * L, L)]
                        nbr_v[e, pl.ds(j * L, L)] = qv * nv
                    return carry2

                lax.fori_loop(0, CHUNK, edge_body, 0)
                pltpu.sync_copy(nbr_v, acc_sh.at[i1_v], add=True)

            return carry

        lax.fori_loop(0, KMAX, chunk_body, 0)
        plsc.subcore_barrier()

        @pl.when(s < NS - 1)
        def _():
            pltpu.sync_copy(
                acc_sh.at[pl.ds(row0, RPT)], out_hbm.at[c].at[pl.ds(row0, RPT)]
            )

        @pl.when(s == NS - 1)
        def _():
            pltpu.sync_copy(
                acc_sh.at[pl.ds(row0, N - (NS - 1) * RPT)],
                out_hbm.at[c].at[pl.ds(row0, N - (NS - 1) * RPT)],
            )

    return k(qt, nf, i0, i1, zc)


def _combine_body(p0_ref, p1_ref, out_ref):
    sm = p0_ref[...] + p1_ref[...]
    out_ref[...] = jnp.where(sm > 0, sm, ALPHA * sm)


def _combine_stage(p0, p1):
    return pl.pallas_call(
        _combine_body,
        grid=(N // NODE_BLK,),
        in_specs=[
            pl.BlockSpec((NODE_BLK, D), lambda i: (i, 0)),
            pl.BlockSpec((NODE_BLK, D), lambda i: (i, 0)),
        ],
        out_specs=pl.BlockSpec((NODE_BLK, D), lambda i: (i, 0)),
        out_shape=jax.ShapeDtypeStruct((N, D), jnp.float32),
    )(p0, p1)


def kernel(node_features, edge_index, edge_features, Wu, bu, a_w, We, be, Wm):
    idx = edge_index.astype(jnp.int32)
    i0 = idx[:, 0]
    i1 = idx[:, 1]
    a1 = a_w[:D, 0].reshape(1, D)
    a2 = a_w[D:, 0].reshape(1, D)

    sab = _node_stage(node_features, Wu, bu.reshape(1, D), a1, a2)
    att = _att_stage(sab[:, 0, :].reshape(N), sab[:, 1, :].reshape(N), i0, i1)
    qt = _edge_stage(
        att.reshape(1, E),
        edge_features.T,
        We.T,
        be.reshape(D, 1),
        Wm.T,
    )
    zc = jnp.zeros((RPT, D), jnp.float32)
    part = _scatter_stage(qt, node_features, i0, i1, zc)
    return _combine_stage(part[0], part[1])


# speedup vs baseline: 3.1262x; 1.1881x over previous
import functools

import jax
import jax.numpy as jnp
from jax import lax
from jax.experimental import pallas as pl
from jax.experimental.pallas import tpu as pltpu
from jax.experimental.pallas import tpu_sc as plsc

N = 10000
E = 320000
D = 128
DE = 16
ALPHA = 0.3

NC = 2
NS = 16
L = 16
NW = NC * NS
EPW = E // NW
CHUNK = 128
NCHT = E // CHUNK
NCHP = 2560
CPW = NCHP // NW
DH = D // 2
ROWS = 632
LROWS = N - (NS - 1) * ROWS

NODE_BLK = 1000
EDGE_BLK = 3200


def _mesh():
    return plsc.VectorSubcoreMesh(
        core_axis_name="c", subcore_axis_name="s", num_cores=NC, num_subcores=NS
    )


def _node_body(nf_ref, wu_ref, bu_ref, a1_ref, a2_ref, sab_ref):
    h = jnp.dot(nf_ref[...], wu_ref[...], preferred_element_type=jnp.float32)
    h = h + bu_ref[...]
    h = jnp.where(h > 0, h, ALPHA * h)
    sa = jnp.sum(h * a1_ref[...], axis=1)
    sb = jnp.sum(h * a2_ref[...], axis=1)
    sab_ref[...] = jnp.stack([sa, sb])[None]


def _node_stage(nf, Wu, bu2, a1, a2):
    return pl.pallas_call(
        _node_body,
        grid=(N // NODE_BLK,),
        in_specs=[
            pl.BlockSpec((NODE_BLK, D), lambda i: (i, 0)),
            pl.BlockSpec((D, D), lambda i: (0, 0)),
            pl.BlockSpec((1, D), lambda i: (0, 0)),
            pl.BlockSpec((1, D), lambda i: (0, 0)),
            pl.BlockSpec((1, D), lambda i: (0, 0)),
        ],
        out_specs=pl.BlockSpec((1, 2, NODE_BLK), lambda i: (i, 0, 0)),
        out_shape=jax.ShapeDtypeStruct((N // NODE_BLK, 2, NODE_BLK), jnp.float32),
    )(nf, Wu, bu2, a1, a2)


def _att_stage(sa, sb, i0, i1):
    @functools.partial(
        pl.kernel,
        out_type=jax.ShapeDtypeStruct((E,), jnp.float32),
        mesh=_mesh(),
        compiler_params=pltpu.CompilerParams(needs_layout_passes=False),
        scratch_types=[
            pltpu.VMEM((N,), jnp.float32),
            pltpu.VMEM((N,), jnp.float32),
            pltpu.VMEM((EPW,), jnp.int32),
            pltpu.VMEM((EPW,), jnp.int32),
            pltpu.VMEM((EPW,), jnp.float32),
        ],
    )
    def k(sa_hbm, sb_hbm, i0_hbm, i1_hbm, att_hbm, sa_v, sb_v, i0_v, i1_v, att_v):
        wid = lax.axis_index("s") * NC + lax.axis_index("c")
        base = wid * EPW
        pltpu.sync_copy(sa_hbm, sa_v)
        pltpu.sync_copy(sb_hbm, sb_v)
        pltpu.sync_copy(i0_hbm.at[pl.ds(base, EPW)], i0_v)
        pltpu.sync_copy(i1_hbm.at[pl.ds(base, EPW)], i1_v)

        def body(kk, carry):
            o = kk * L
            va = plsc.load_gather(sa_v, [i1_v[pl.ds(o, L)]])
            vb = plsc.load_gather(sb_v, [i0_v[pl.ds(o, L)]])
            att_v[pl.ds(o, L)] = va + vb
            return carry

        lax.fori_loop(0, EPW // L, body, 0)
        pltpu.sync_copy(att_v, att_hbm.at[pl.ds(base, EPW)])

    return k(sa, sb, i0, i1)


def _edge_body(att_ref, eft_ref, wet_ref, bet_ref, wmt_ref, qt_ref):
    ut = jnp.dot(wet_ref[...], eft_ref[...], preferred_element_type=jnp.float32)
    ut = ut + bet_ref[...]
    xt = ut * att_ref[...]
    xt = xt - jnp.max(xt, axis=0, keepdims=True)
    p = jnp.exp(xt)
    p = p / jnp.sum(p, axis=0, keepdims=True)
    yt = jnp.dot(wmt_ref[...], p, preferred_element_type=jnp.float32)
    yt = yt - jnp.max(yt, axis=0, keepdims=True)
    q = jnp.exp(yt)
    qt_ref[...] = q / jnp.sum(q, axis=0, keepdims=True)


def _edge_stage(att2d, eft, wet, bet, wmt):
    return pl.pallas_call(
        _edge_body,
        grid=(E // EDGE_BLK,),
        in_specs=[
            pl.BlockSpec((1, EDGE_BLK), lambda i: (0, i)),
            pl.BlockSpec((DE, EDGE_BLK), lambda i: (0, i)),
            pl.BlockSpec((D, DE), lambda i: (0, 0)),
            pl.BlockSpec((D, 1), lambda i: (0, 0)),
            pl.BlockSpec((D, D), lambda i: (0, 0)),
        ],
        out_specs=pl.BlockSpec((D, EDGE_BLK), lambda i: (0, i)),
        out_shape=jax.ShapeDtypeStruct((D, E), jnp.float32),
    )(att2d, eft, wet, bet, wmt)


def _scatter_stage(qt, nf, i0f, i1f, zc):
    @functools.partial(
        pl.kernel,
        out_type=jax.ShapeDtypeStruct((NC, N, D), jnp.float32),
        mesh=_mesh(),
        compiler_params=pltpu.CompilerParams(needs_layout_passes=False),
        scratch_types=[
            pltpu.VMEM((4, CHUNK), jnp.int32),
            pltpu.VMEM((2, CHUNK), jnp.int32),
            pltpu.VMEM((CHUNK, D), jnp.float32),
            pltpu.VMEM((CHUNK, D), jnp.float32),
            pltpu.VMEM((DH, CHUNK), jnp.float32),
            pltpu.VMEM((DH, CHUNK), jnp.float32),
            pltpu.VMEM_SHARED((N, D), jnp.float32),
            pltpu.SemaphoreType.DMA,
            pltpu.SemaphoreType.DMA,
            pltpu.SemaphoreType.DMA,
            pltpu.SemaphoreType.DMA,
            pltpu.SemaphoreType.DMA,
            pltpu.SemaphoreType.DMA,
        ],
    )
    def k(qt_hbm, nf_hbm, i0_hbm, i1_hbm, zc_hbm, out_hbm,
          i0r, i1r, nbr0, nbr1, qta, qtb, acc_sh,
          sg0, sg1, sqa, sqb, si0, si1):
        c = lax.axis_index("c")
        s = lax.axis_index("s")
        w = s * NC + c
        row0 = pl.multiple_of(s * ROWS, 8)

        @pl.when(s < NS - 1)
        def _():
            pltpu.sync_copy(zc_hbm.at[pl.ds(0, ROWS)],
                            acc_sh.at[pl.ds(row0, ROWS)])

        @pl.when(s == NS - 1)
        def _():
            pltpu.sync_copy(zc_hbm.at[pl.ds(0, LROWS)],
                            acc_sh.at[pl.ds(row0, LROWS)])

        plsc.subcore_barrier()

        iot = lax.iota(jnp.int32, L)
        chunk0 = w * CPW
        nvalid = jnp.minimum(NCHT - chunk0, CPW)
        nbrs = (nbr0, nbr1)
        sgs = (sg0, sg1)

        def ebase(g):
            return pl.multiple_of((chunk0 + g) * CHUNK, CHUNK)

        def start_i0(g, slot):
            pltpu.async_copy(i0_hbm.at[pl.ds(ebase(g), CHUNK)],
                             i0r.at[slot], si0)

        def start_i1(g, slot):
            pltpu.async_copy(i1_hbm.at[pl.ds(ebase(g), CHUNK)],
                             i1r.at[slot], si1)

        def start_gather(g, b, slot):
            pltpu.async_copy(nf_hbm.at[i0r.at[slot]], nbrs[b], sgs[b])

        def start_qt(g, half, dst, sem):
            r = pl.multiple_of(half * DH, DH)
            pltpu.async_copy(
                qt_hbm.at[pl.ds(r, DH), pl.ds(ebase(g), CHUNK)], dst, sem)

        def drain(src, dst, sem):
            pltpu.make_async_copy(src, dst, sem).wait()

        def compute_half(b, qref, j0):
            def edge_body(e, carry):
                col = jnp.zeros((L,), jnp.int32) + e
                for j in range(DH // L):
                    qv = plsc.load_gather(qref, [iot + j * L, col])
                    sl = pl.ds((j0 + j) * L, L)
                    nbrs[b][e, sl] = qv * nbrs[b][e, sl]
                return carry

            lax.fori_loop(0, CHUNK, edge_body, 0)

        pltpu.sync_copy(i0_hbm.at[pl.ds(ebase(0), CHUNK)], i0r.at[0])
        pltpu.sync_copy(i0_hbm.at[pl.ds(ebase(1), CHUNK)], i0r.at[1])
        pltpu.sync_copy(i0_hbm.at[pl.ds(ebase(2), CHUNK)], i0r.at[2])
        pltpu.sync_copy(i1_hbm.at[pl.ds(ebase(0), CHUNK)], i1r.at[0])
        start_gather(0, 0, 0)
        start_gather(1, 1, 1)
        start_qt(0, 0, qta, sqa)
        start_qt(0, 1, qtb, sqb)

        def outer(tt, carry):
            for bb in range(4):
                g = 4 * tt + bb
                b = bb % 2
                s0 = (bb + 3) % 4
                s1 = (bb + 1) % 2
                sg2 = (bb + 2) % 4

                @pl.when(g < nvalid)
                def _(g=g, b=b, s0=s0, s1=s1, sg2=sg2):
                    @pl.when(g + 3 < nvalid)
                    def _():
                        start_i0(g + 3, s0)

                    @pl.when(g + 1 < nvalid)
                    def _():
                        start_i1(g + 1, s1)

                    drain(nf_hbm.at[pl.ds(0, CHUNK)], nbrs[b], sgs[b])
                    drain(qt_hbm.at[pl.ds(0, DH), pl.ds(0, CHUNK)], qta, sqa)
                    compute_half(b, qta, 0)

                    @pl.when(g + 1 < nvalid)
                    def _():
                        start_qt(g + 1, 0, qta, sqa)

                    drain(qt_hbm.at[pl.ds(0, DH), pl.ds(0, CHUNK)], qtb, sqb)
                    compute_half(b, qtb, DH // L)

                    @pl.when(g + 1 < nvalid)
                    def _():
                        start_qt(g + 1, 1, qtb, sqb)

                    @pl.when(g >= 1)
                    def _():
                        drain(i1_hbm.at[pl.ds(0, CHUNK)], i1r.at[0], si1)

                    pltpu.sync_copy(nbrs[b], acc_sh.at[i1r.at[bb % 2]],
                                    add=True)

                    @pl.when(g + 2 < nvalid)
                    def _():
                        @pl.when(g >= 1)
                        def _():
                            drain(i0_hbm.at[pl.ds(0, CHUNK)], i0r.at[0], si0)

                        start_gather(g + 2, b, sg2)

            return carry

        lax.fori_loop(0, CPW // 4, outer, 0)
        plsc.subcore_barrier()

        @pl.when(s < NS - 1)
        def _():
            pltpu.sync_copy(acc_sh.at[pl.ds(row0, ROWS)],
                            out_hbm.at[c].at[pl.ds(row0, ROWS)])

        @pl.when(s == NS - 1)
        def _():
            pltpu.sync_copy(acc_sh.at[pl.ds(row0, LROWS)],
                            out_hbm.at[c].at[pl.ds(row0, LROWS)])

    return k(qt, nf, i0f, i1f, zc)


def _combine_body(p0_ref, p1_ref, out_ref):
    sm = p0_ref[...] + p1_ref[...]
    out_ref[...] = jnp.where(sm > 0, sm, ALPHA * sm)


def _combine_stage(p0, p1):
    return pl.pallas_call(
        _combine_body,
        grid=(N // NODE_BLK,),
        in_specs=[
            pl.BlockSpec((NODE_BLK, D), lambda i: (i, 0)),
            pl.BlockSpec((NODE_BLK, D), lambda i: (i, 0)),
        ],
        out_specs=pl.BlockSpec((NODE_BLK, D), lambda i: (i, 0)),
        out_shape=jax.ShapeDtypeStruct((N, D), jnp.float32),
    )(p0, p1)


def kernel(node_features, edge_index, edge_features, Wu, bu, a_w, We, be, Wm):
    idx = edge_index.astype(jnp.int32)
    i0 = idx[:, 0]
    i1 = idx[:, 1]
    a1 = a_w[:D, 0].reshape(1, D)
    a2 = a_w[D:, 0].reshape(1, D)

    sab = _node_stage(node_features, Wu, bu.reshape(1, D), a1, a2)
    att = _att_stage(sab[:, 0, :].reshape(N), sab[:, 1, :].reshape(N), i0, i1)
    qt = _edge_stage(
        att.reshape(1, E),
        edge_features.T,
        We.T,
        be.reshape(D, 1),
        Wm.T,
    )
    pad = jnp.zeros((NCHP * CHUNK - E,), jnp.int32)
    i0f = jnp.concatenate([i0, pad])
    i1f = jnp.concatenate([i1, pad])
    zc = jnp.zeros((ROWS, D), jnp.float32)
    part = _scatter_stage(qt, node_features, i0f, i1f, zc)
    return _combine_stage(part[0], part[1])


# speedup vs baseline: 7.7072x; 2.4654x over previous
import functools

import jax
import jax.numpy as jnp
from jax import lax
from jax.experimental import pallas as pl
from jax.experimental.pallas import tpu as pltpu
from jax.experimental.pallas import tpu_sc as plsc

N = 10000
E = 320000
D = 128
DE = 16
ALPHA = 0.3

NC = 2
NS = 16
L = 16
NW = NC * NS
EPW = E // NW
CHUNK = 128
NCHT = E // CHUNK
NCHP = 2560
CPW = NCHP // NW
DH = D // 2
ROWS = 632
LROWS = N - (NS - 1) * ROWS

NODE_BLK = 1000
EDGE_BLK = 3200


def _mesh():
    return plsc.VectorSubcoreMesh(
        core_axis_name="c", subcore_axis_name="s", num_cores=NC, num_subcores=NS
    )


def _node_body(nf_ref, wu_ref, bu_ref, a1_ref, a2_ref, sab_ref):
    h = jnp.dot(nf_ref[...], wu_ref[...], preferred_element_type=jnp.float32)
    h = h + bu_ref[...]
    h = jnp.where(h > 0, h, ALPHA * h)
    sa = jnp.sum(h * a1_ref[...], axis=1)
    sb = jnp.sum(h * a2_ref[...], axis=1)
    sab_ref[...] = jnp.stack([sa, sb])[None]


def _node_stage(nf, Wu, bu2, a1, a2):
    return pl.pallas_call(
        _node_body,
        grid=(N // NODE_BLK,),
        in_specs=[
            pl.BlockSpec((NODE_BLK, D), lambda i: (i, 0)),
            pl.BlockSpec((D, D), lambda i: (0, 0)),
            pl.BlockSpec((1, D), lambda i: (0, 0)),
            pl.BlockSpec((1, D), lambda i: (0, 0)),
            pl.BlockSpec((1, D), lambda i: (0, 0)),
        ],
        out_specs=pl.BlockSpec((1, 2, NODE_BLK), lambda i: (i, 0, 0)),
        out_shape=jax.ShapeDtypeStruct((N // NODE_BLK, 2, NODE_BLK), jnp.float32),
    )(nf, Wu, bu2, a1, a2)


def _att_stage(sa, sb, i0, i1):
    @functools.partial(
        pl.kernel,
        out_type=jax.ShapeDtypeStruct((E,), jnp.float32),
        mesh=_mesh(),
        compiler_params=pltpu.CompilerParams(needs_layout_passes=False),
        scratch_types=[
            pltpu.VMEM((N,), jnp.float32),
            pltpu.VMEM((N,), jnp.float32),
            pltpu.VMEM((EPW,), jnp.int32),
            pltpu.VMEM((EPW,), jnp.int32),
            pltpu.VMEM((EPW,), jnp.float32),
        ],
    )
    def k(sa_hbm, sb_hbm, i0_hbm, i1_hbm, att_hbm, sa_v, sb_v, i0_v, i1_v, att_v):
        wid = lax.axis_index("s") * NC + lax.axis_index("c")
        base = wid * EPW
        pltpu.sync_copy(sa_hbm, sa_v)
        pltpu.sync_copy(sb_hbm, sb_v)
        pltpu.sync_copy(i0_hbm.at[pl.ds(base, EPW)], i0_v)
        pltpu.sync_copy(i1_hbm.at[pl.ds(base, EPW)], i1_v)

        def body(kk, carry):
            o = kk * L
            va = plsc.load_gather(sa_v, [i1_v[pl.ds(o, L)]])
            vb = plsc.load_gather(sb_v, [i0_v[pl.ds(o, L)]])
            att_v[pl.ds(o, L)] = va + vb
            return carry

        lax.fori_loop(0, EPW // L, body, 0)
        pltpu.sync_copy(att_v, att_hbm.at[pl.ds(base, EPW)])

    return k(sa, sb, i0, i1)


def _edge_body(att_ref, ef_ref, we_ref, be_ref, wm_ref, q_ref):
    u = jnp.dot(ef_ref[...], we_ref[...], preferred_element_type=jnp.float32)
    u = u + be_ref[...]
    x = u * att_ref[...].reshape(EDGE_BLK, 1)
    x = x - jnp.max(x, axis=1, keepdims=True)
    p = jnp.exp(x)
    p = p / jnp.sum(p, axis=1, keepdims=True)
    y = jnp.dot(p, wm_ref[...], preferred_element_type=jnp.float32)
    y = y - jnp.max(y, axis=1, keepdims=True)
    q = jnp.exp(y)
    q_ref[...] = q / jnp.sum(q, axis=1, keepdims=True)


def _edge_stage(att2d, ef, we, be2, wm):
    return pl.pallas_call(
        _edge_body,
        grid=(E // EDGE_BLK,),
        in_specs=[
            pl.BlockSpec((1, EDGE_BLK), lambda i: (0, i)),
            pl.BlockSpec((EDGE_BLK, DE), lambda i: (i, 0)),
            pl.BlockSpec((DE, D), lambda i: (0, 0)),
            pl.BlockSpec((1, D), lambda i: (0, 0)),
            pl.BlockSpec((D, D), lambda i: (0, 0)),
        ],
        out_specs=pl.BlockSpec((EDGE_BLK, D), lambda i: (i, 0)),
        out_shape=jax.ShapeDtypeStruct((E, D), jnp.float32),
    )(att2d, ef, we, be2, wm)


def _scatter_stage(qt, nf, i0f, i1f, zc):
    @functools.partial(
        pl.kernel,
        out_type=jax.ShapeDtypeStruct((NC, N, D), jnp.float32),
        mesh=_mesh(),
        compiler_params=pltpu.CompilerParams(needs_layout_passes=False),
        scratch_types=[
            pltpu.VMEM((4, CHUNK), jnp.int32),
            pltpu.VMEM((2, CHUNK), jnp.int32),
            pltpu.VMEM((CHUNK, D), jnp.float32),
            pltpu.VMEM((CHUNK, D), jnp.float32),
            pltpu.VMEM((CHUNK // 2, D), jnp.float32),
            pltpu.VMEM((CHUNK // 2, D), jnp.float32),
            pltpu.VMEM_SHARED((N, D), jnp.float32),
            pltpu.SemaphoreType.DMA,
            pltpu.SemaphoreType.DMA,
            pltpu.SemaphoreType.DMA,
            pltpu.SemaphoreType.DMA,
            pltpu.SemaphoreType.DMA,
            pltpu.SemaphoreType.DMA,
        ],
    )
    def k(qt_hbm, nf_hbm, i0_hbm, i1_hbm, zc_hbm, out_hbm,
          i0r, i1r, nbr0, nbr1, qta, qtb, acc_sh,
          sg0, sg1, sqa, sqb, si0, si1):
        c = lax.axis_index("c")
        s = lax.axis_index("s")
        w = s * NC + c
        row0 = pl.multiple_of(s * ROWS, 8)

        @pl.when(s < NS - 1)
        def _():
            pltpu.sync_copy(zc_hbm.at[pl.ds(0, ROWS)],
                            acc_sh.at[pl.ds(row0, ROWS)])

        @pl.when(s == NS - 1)
        def _():
            pltpu.sync_copy(zc_hbm.at[pl.ds(0, LROWS)],
                            acc_sh.at[pl.ds(row0, LROWS)])

        plsc.subcore_barrier()

        iot = lax.iota(jnp.int32, L)
        chunk0 = w * CPW
        nvalid = jnp.minimum(NCHT - chunk0, CPW)
        nbrs = (nbr0, nbr1)
        sgs = (sg0, sg1)

        def ebase(g):
            return pl.multiple_of((chunk0 + g) * CHUNK, CHUNK)

        def start_i0(g, slot):
            pltpu.async_copy(i0_hbm.at[pl.ds(ebase(g), CHUNK)],
                             i0r.at[slot], si0)

        def start_i1(g, slot):
            pltpu.async_copy(i1_hbm.at[pl.ds(ebase(g), CHUNK)],
                             i1r.at[slot], si1)

        def start_gather(g, b, slot):
            pltpu.async_copy(nf_hbm.at[i0r.at[slot]], nbrs[b], sgs[b])

        def start_qt(g, half, dst, sem):
            r = pl.multiple_of(ebase(g) + half * (CHUNK // 2), CHUNK // 2)
            pltpu.async_copy(qt_hbm.at[pl.ds(r, CHUNK // 2)], dst, sem)

        def drain(src, dst, sem):
            pltpu.make_async_copy(src, dst, sem).wait()

        def compute_half(b, qref, e0):
            def edge_body(e, carry):
                for j in range(D // L):
                    sl = pl.ds(j * L, L)
                    qv = qref[e, sl]
                    nbrs[b][e0 + e, sl] = qv * nbrs[b][e0 + e, sl]
                return carry

            lax.fori_loop(0, CHUNK // 2, edge_body, 0)

        pltpu.sync_copy(i0_hbm.at[pl.ds(ebase(0), CHUNK)], i0r.at[0])
        pltpu.sync_copy(i0_hbm.at[pl.ds(ebase(1), CHUNK)], i0r.at[1])
        pltpu.sync_copy(i0_hbm.at[pl.ds(ebase(2), CHUNK)], i0r.at[2])
        pltpu.sync_copy(i1_hbm.at[pl.ds(ebase(0), CHUNK)], i1r.at[0])
        start_gather(0, 0, 0)
        start_gather(1, 1, 1)
        start_qt(0, 0, qta, sqa)
        start_qt(0, 1, qtb, sqb)

        def outer(tt, carry):
            for bb in range(4):
                g = 4 * tt + bb
                b = bb % 2
                s0 = (bb + 3) % 4
                s1 = (bb + 1) % 2
                sg2 = (bb + 2) % 4

                @pl.when(g < nvalid)
                def _(g=g, b=b, s0=s0, s1=s1, sg2=sg2):
                    @pl.when(g + 3 < nvalid)
                    def _():
                        start_i0(g + 3, s0)

                    @pl.when(g + 1 < nvalid)
                    def _():
                        start_i1(g + 1, s1)

                    drain(nf_hbm.at[pl.ds(0, CHUNK)], nbrs[b], sgs[b])
                    drain(qt_hbm.at[pl.ds(0, CHUNK // 2)], qta, sqa)
                    compute_half(b, qta, 0)

                    @pl.when(g + 1 < nvalid)
                    def _():
                        start_qt(g + 1, 0, qta, sqa)

                    drain(qt_hbm.at[pl.ds(0, CHUNK // 2)], qtb, sqb)
                    compute_half(b, qtb, CHUNK // 2)

                    @pl.when(g + 1 < nvalid)
                    def _():
                        start_qt(g + 1, 1, qtb, sqb)

                    @pl.when(g >= 1)
                    def _():
                        drain(i1_hbm.at[pl.ds(0, CHUNK)], i1r.at[0], si1)

                    pltpu.sync_copy(nbrs[b], acc_sh.at[i1r.at[bb % 2]],
                                    add=True)

                    @pl.when(g + 2 < nvalid)
                    def _():
                        @pl.when(g >= 1)
                        def _():
                            drain(i0_hbm.at[pl.ds(0, CHUNK)], i0r.at[0], si0)

                        start_gather(g + 2, b, sg2)

            return carry

        lax.fori_loop(0, CPW // 4, outer, 0)
        plsc.subcore_barrier()

        @pl.when(s < NS - 1)
        def _():
            pltpu.sync_copy(acc_sh.at[pl.ds(row0, ROWS)],
                            out_hbm.at[c].at[pl.ds(row0, ROWS)])

        @pl.when(s == NS - 1)
        def _():
            pltpu.sync_copy(acc_sh.at[pl.ds(row0, LROWS)],
                            out_hbm.at[c].at[pl.ds(row0, LROWS)])

    return k(qt, nf, i0f, i1f, zc)


def _combine_body(p0_ref, p1_ref, out_ref):
    sm = p0_ref[...] + p1_ref[...]
    out_ref[...] = jnp.where(sm > 0, sm, ALPHA * sm)


def _combine_stage(p0, p1):
    return pl.pallas_call(
        _combine_body,
        grid=(N // NODE_BLK,),
        in_specs=[
            pl.BlockSpec((NODE_BLK, D), lambda i: (i, 0)),
            pl.BlockSpec((NODE_BLK, D), lambda i: (i, 0)),
        ],
        out_specs=pl.BlockSpec((NODE_BLK, D), lambda i: (i, 0)),
        out_shape=jax.ShapeDtypeStruct((N, D), jnp.float32),
    )(p0, p1)


def kernel(node_features, edge_index, edge_features, Wu, bu, a_w, We, be, Wm):
    idx = edge_index.astype(jnp.int32)
    i0 = idx[:, 0]
    i1 = idx[:, 1]
    a1 = a_w[:D, 0].reshape(1, D)
    a2 = a_w[D:, 0].reshape(1, D)

    sab = _node_stage(node_features, Wu, bu.reshape(1, D), a1, a2)
    att = _att_stage(sab[:, 0, :].reshape(N), sab[:, 1, :].reshape(N), i0, i1)
    qt = _edge_stage(
        att.reshape(1, E),
        edge_features,
        We,
        be.reshape(1, D),
        Wm,
    )
    pad = jnp.zeros((NCHP * CHUNK - E,), jnp.int32)
    i0f = jnp.concatenate([i0, pad])
    i1f = jnp.concatenate([i1, pad])
    zc = jnp.zeros((ROWS, D), jnp.float32)
    part = _scatter_stage(qt, node_features, i0f, i1f, zc)
    return _combine_stage(part[0], part[1])
